# Initial kernel scaffold; baseline (speedup 1.0000x reference)
import jax
import jax.numpy as jnp
from jax.experimental import pallas as pl


def kernel(x, sem_proj_w, epi_proj_w, sem_readout_w, epi_readout_w, null_gate_w, null_gate_b, out_proj_w, out_proj_b, ln_gamma, ln_beta, W_res_sem, W_in_sem, W_res_epi, W_in_epi):
    raise NotImplementedError("write your pallas kernel here")


# speedup vs baseline: 3.8901x; 3.8901x over previous
import functools

import jax
import jax.numpy as jnp
from jax import lax
from jax.experimental import pallas as pl
from jax.experimental.pallas import tpu as pltpu

GATE_THRESH = 0.7
ALPHA_REF = 0.95
L_REF = 96.0
EPS = 1e-6
_STEPS_PER_ITER = 4


def _gate_kernel(xw_ref, pcat_ref, winTs_ref, winTe_ref,
                 us_ref, ue_ref, fire_ref,
                 kk_ref, kin_s_ref, kin_e_ref, ema_s_ref, ema_e_ref,
                 *, T, B, H, L, a):
    kk_ref[...] = jnp.dot(xw_ref[...], pcat_ref[...],
                          preferred_element_type=jnp.float32)
    ema_s_ref[...] = jnp.zeros_like(ema_s_ref)
    ema_e_ref[...] = jnp.zeros_like(ema_e_ref)
    S = _STEPS_PER_ITER
    rows_total = kin_s_ref.shape[0]
    n_iters = rows_total // B // S

    def body(i, carry):
        rows = pl.ds(S * B * i, S * B)
        kk_blk = kk_ref[rows, :]
        ema_s = ema_s_ref[...]
        ema_e = ema_e_ref[...]
        kin_s_parts, kin_e_parts, fire_parts = [], [], []
        for j in range(S):
            ks_t = kk_blk[B * j:B * (j + 1), :H]
            ke_t = kk_blk[B * j:B * (j + 1), H:]
            norm_s = jnp.sqrt(jnp.sum(ks_t * ks_t, axis=-1, keepdims=True))
            norm_e = jnp.sqrt(jnp.sum(ke_t * ke_t, axis=-1, keepdims=True))
            d_s = ks_t - ema_s
            d_e = ke_t - ema_e
            err_s = jnp.sqrt(jnp.sum(d_s * d_s, axis=-1, keepdims=True))
            err_e = jnp.sqrt(jnp.sum(d_e * d_e, axis=-1, keepdims=True))
            fire = ((err_s >= GATE_THRESH * norm_s) |
                    (err_e >= GATE_THRESH * norm_e))
            tg = S * i + j
            fire_f = jnp.where(tg < T, fire.astype(jnp.float32), 0.0)
            ema_s = a * ema_s + (1.0 - a) * ks_t
            ema_e = a * ema_e + (1.0 - a) * ke_t
            w_t = (tg + 1).astype(jnp.float32) / L
            kin_s_parts.append(ks_t / jnp.maximum(norm_s, EPS))
            kin_e_parts.append(w_t * (ke_t / jnp.maximum(norm_e, EPS)))
            fire_parts.append(fire_f)
        ema_s_ref[...] = ema_s
        ema_e_ref[...] = ema_e
        kin_s_ref[rows, :] = jnp.concatenate(kin_s_parts, axis=0)
        kin_e_ref[rows, :] = jnp.concatenate(kin_e_parts, axis=0)
        fire_ref[rows, :] = jnp.concatenate(fire_parts, axis=0)
        return carry

    lax.fori_loop(0, n_iters, body, 0)

    bf16 = jnp.bfloat16
    chunk = min(256, rows_total)
    for c in range(0, rows_total, chunk):
        sl = pl.ds(c, chunk)
        us_ref[sl, :] = jnp.dot(
            kin_s_ref[sl, :].astype(bf16), winTs_ref[...],
            preferred_element_type=jnp.float32).astype(bf16)
        ue_ref[sl, :] = jnp.dot(
            kin_e_ref[sl, :].astype(bf16), winTe_ref[...],
            preferred_element_type=jnp.float32).astype(bf16)


def _res_kernel(wres_ref, u_ref, fire_ref, hout_ref, h_ref, *, n_iters):
    i = pl.program_id(0)

    @pl.when(i == 0)
    def _():
        h_ref[...] = jnp.zeros_like(h_ref)

    h = h_ref[...]
    pre = jnp.dot(h.astype(jnp.bfloat16), wres_ref[...],
                  preferred_element_type=jnp.float32)
    hn = jnp.tanh(pre + u_ref[0].astype(jnp.float32))
    h_ref[...] = jnp.where(fire_ref[0] > 0.5, hn, h)

    @pl.when(i == n_iters - 1)
    def _():
        hout_ref[...] = h_ref[...]


def _out_kernel(hs_ref, he_ref, roS_ref, roE_ref, xq_ref, ngw_ref, ngb_ref,
                opw_ref, opb_ref, gam_ref, bet_ref, y_ref):
    r1 = jnp.dot(hs_ref[...], roS_ref[...], preferred_element_type=jnp.float32)
    r2 = jnp.dot(he_ref[...], roE_ref[...], preferred_element_type=jnp.float32)
    r = jnp.concatenate([r1, r2], axis=-1)
    g = jax.nn.sigmoid(jnp.dot(xq_ref[...], ngw_ref[...],
                               preferred_element_type=jnp.float32)
                       + ngb_ref[...])
    y = jnp.dot(g * r, opw_ref[...], preferred_element_type=jnp.float32)
    y = y + opb_ref[...]
    mu = jnp.mean(y, axis=-1, keepdims=True)
    yc = y - mu
    var = jnp.mean(yc * yc, axis=-1, keepdims=True)
    y_ref[...] = yc / jnp.sqrt(var + 1e-5) * gam_ref[...] + bet_ref[...]


def kernel(x, sem_proj_w, epi_proj_w, sem_readout_w, epi_readout_w,
           null_gate_w, null_gate_b, out_proj_w, out_proj_b,
           ln_gamma, ln_beta, W_res_sem, W_in_sem, W_res_epi, W_in_epi):
    B, L, D = x.shape
    T = L - 1
    S = _STEPS_PER_ITER
    TP = ((T + S - 1) // S) * S
    H = sem_proj_w.shape[0]
    N = W_res_sem.shape[0]
    a = ALPHA_REF ** (L_REF / L)
    f32 = jnp.float32
    bf16 = jnp.bfloat16

    xw = jnp.transpose(x[:, :-1, :], (1, 0, 2)).reshape(T * B, D)
    xw = jnp.pad(xw, ((0, (TP - T) * B), (0, 0)))
    pcat = jnp.concatenate([sem_proj_w.T, epi_proj_w.T], axis=1)

    u_s, u_e, fire = pl.pallas_call(
        functools.partial(_gate_kernel, T=T, B=B, H=H, L=L, a=a),
        out_shape=[
            jax.ShapeDtypeStruct((TP * B, N), bf16),
            jax.ShapeDtypeStruct((TP * B, N), bf16),
            jax.ShapeDtypeStruct((TP * B, 1), f32),
        ],
        scratch_shapes=[
            pltpu.VMEM((TP * B, 2 * H), f32),
            pltpu.VMEM((TP * B, H), f32),
            pltpu.VMEM((TP * B, H), f32),
            pltpu.VMEM((B, H), f32),
            pltpu.VMEM((B, H), f32),
        ],
        compiler_params=pltpu.CompilerParams(
            vmem_limit_bytes=56 * 1024 * 1024),
        name="esn_gate",
    )(xw, pcat, W_in_sem.T.astype(bf16), W_in_epi.T.astype(bf16))

    u_s = u_s.reshape(TP, B, N)
    u_e = u_e.reshape(TP, B, N)
    fire3 = fire.reshape(TP, B, 1)
    res_call = pl.pallas_call(
        functools.partial(_res_kernel, n_iters=TP),
        grid=(TP,),
        in_specs=[
            pl.BlockSpec((N, N), lambda i: (0, 0)),
            pl.BlockSpec((1, B, N), lambda i: (i, 0, 0)),
            pl.BlockSpec((1, B, 1), lambda i: (i, 0, 0)),
        ],
        out_specs=pl.BlockSpec((B, N), lambda i: (0, 0)),
        out_shape=jax.ShapeDtypeStruct((B, N), f32),
        scratch_shapes=[pltpu.VMEM((B, N), f32)],
        compiler_params=pltpu.CompilerParams(
            dimension_semantics=("arbitrary",),
            vmem_limit_bytes=40 * 1024 * 1024),
        name="esn_reservoir",
    )
    h_s = res_call(W_res_sem.T.astype(bf16), u_s, fire3)
    h_e = res_call(W_res_epi.T.astype(bf16), u_e, fire3)

    y = pl.pallas_call(
        _out_kernel,
        out_shape=jax.ShapeDtypeStruct((B, D), f32),
        name="esn_out",
    )(h_s, h_e, sem_readout_w.T, epi_readout_w.T, x[:, -1, :],
      null_gate_w.T, null_gate_b.reshape(1, 1), out_proj_w.T,
      out_proj_b.reshape(1, D), ln_gamma.reshape(1, D), ln_beta.reshape(1, D))
    return y


# speedup vs baseline: 6.7151x; 1.7262x over previous
import functools

import jax
import jax.numpy as jnp
import numpy as np
from jax import lax
from jax.experimental import pallas as pl
from jax.experimental.pallas import tpu as pltpu
from jax.sharding import Mesh, PartitionSpec as P

GATE_THRESH = 0.7
ALPHA_REF = 0.95
L_REF = 96.0
EPS = 1e-6
_STEPS_PER_ITER = 4


def _gate_kernel(xw_ref, pcat_ref, winT_ref, sel_ref,
                 u_ref, fire_ref,
                 kk_ref, kin_s_ref, kin_e_ref, ema_s_ref, ema_e_ref,
                 *, T, B, H, L, a):
    kk_ref[...] = jnp.dot(xw_ref[...], pcat_ref[...],
                          preferred_element_type=jnp.float32)
    ema_s_ref[...] = jnp.zeros_like(ema_s_ref)
    ema_e_ref[...] = jnp.zeros_like(ema_e_ref)
    S = _STEPS_PER_ITER
    rows_total = kin_s_ref.shape[0]
    n_iters = rows_total // B // S

    def body(i, carry):
        rows = pl.ds(S * B * i, S * B)
        kk_blk = kk_ref[rows, :]
        ema_s = ema_s_ref[...]
        ema_e = ema_e_ref[...]
        kin_s_parts, kin_e_parts, fire_parts = [], [], []
        for j in range(S):
            ks_t = kk_blk[B * j:B * (j + 1), :H]
            ke_t = kk_blk[B * j:B * (j + 1), H:]
            norm_s = jnp.sqrt(jnp.sum(ks_t * ks_t, axis=-1, keepdims=True))
            norm_e = jnp.sqrt(jnp.sum(ke_t * ke_t, axis=-1, keepdims=True))
            d_s = ks_t - ema_s
            d_e = ke_t - ema_e
            err_s = jnp.sqrt(jnp.sum(d_s * d_s, axis=-1, keepdims=True))
            err_e = jnp.sqrt(jnp.sum(d_e * d_e, axis=-1, keepdims=True))
            fire = ((err_s >= GATE_THRESH * norm_s) |
                    (err_e >= GATE_THRESH * norm_e))
            tg = S * i + j
            fire_f = jnp.where(tg < T, fire.astype(jnp.float32), 0.0)
            ema_s = a * ema_s + (1.0 - a) * ks_t
            ema_e = a * ema_e + (1.0 - a) * ke_t
            w_t = (tg + 1).astype(jnp.float32) / L
            kin_s_parts.append(ks_t / jnp.maximum(norm_s, EPS))
            kin_e_parts.append(w_t * (ke_t / jnp.maximum(norm_e, EPS)))
            fire_parts.append(fire_f)
        ema_s_ref[...] = ema_s
        ema_e_ref[...] = ema_e
        kin_s_ref[rows, :] = jnp.concatenate(kin_s_parts, axis=0)
        kin_e_ref[rows, :] = jnp.concatenate(kin_e_parts, axis=0)
        fire_ref[rows, :] = jnp.concatenate(fire_parts, axis=0)
        return carry

    lax.fori_loop(0, n_iters, body, 0)

    bf16 = jnp.bfloat16
    sel = sel_ref[0, 0]
    chunk = min(256, rows_total)
    for c in range(0, rows_total, chunk):
        sl = pl.ds(c, chunk)
        kin = jnp.where(sel > 0.5, kin_s_ref[sl, :], kin_e_ref[sl, :])
        u_ref[sl, :] = jnp.dot(kin.astype(bf16), winT_ref[...],
                               preferred_element_type=jnp.float32
                               ).astype(bf16)


def _res_kernel(wres_ref, u_ref, fire_ref, hout_ref, h_ref, *, n_iters):
    i = pl.program_id(0)

    @pl.when(i == 0)
    def _():
        h_ref[...] = jnp.zeros_like(h_ref)

    h = h_ref[...]
    pre = jnp.dot(h.astype(jnp.bfloat16), wres_ref[...],
                  preferred_element_type=jnp.float32)
    hn = jnp.tanh(pre + u_ref[0].astype(jnp.float32))
    h_ref[...] = jnp.where(fire_ref[0] > 0.5, hn, h)

    @pl.when(i == n_iters - 1)
    def _():
        hout_ref[...] = h_ref[...]


def _out_kernel(hs_ref, he_ref, roS_ref, roE_ref, xq_ref, ngw_ref, ngb_ref,
                opw_ref, opb_ref, gam_ref, bet_ref, y_ref):
    r1 = jnp.dot(hs_ref[...], roS_ref[...], preferred_element_type=jnp.float32)
    r2 = jnp.dot(he_ref[...], roE_ref[...], preferred_element_type=jnp.float32)
    r = jnp.concatenate([r1, r2], axis=-1)
    g = jax.nn.sigmoid(jnp.dot(xq_ref[...], ngw_ref[...],
                               preferred_element_type=jnp.float32)
                       + ngb_ref[...])
    y = jnp.dot(g * r, opw_ref[...], preferred_element_type=jnp.float32)
    y = y + opb_ref[...]
    mu = jnp.mean(y, axis=-1, keepdims=True)
    yc = y - mu
    var = jnp.mean(yc * yc, axis=-1, keepdims=True)
    y_ref[...] = yc / jnp.sqrt(var + 1e-5) * gam_ref[...] + bet_ref[...]


def kernel(x, sem_proj_w, epi_proj_w, sem_readout_w, epi_readout_w,
           null_gate_w, null_gate_b, out_proj_w, out_proj_b,
           ln_gamma, ln_beta, W_res_sem, W_in_sem, W_res_epi, W_in_epi):
    B, L, D = x.shape
    T = L - 1
    S = _STEPS_PER_ITER
    TP = ((T + S - 1) // S) * S
    H = sem_proj_w.shape[0]
    N = W_res_sem.shape[0]
    a = ALPHA_REF ** (L_REF / L)
    f32 = jnp.float32
    bf16 = jnp.bfloat16

    xw = jnp.transpose(x[:, :-1, :], (1, 0, 2)).reshape(T * B, D)
    xw = jnp.pad(xw, ((0, (TP - T) * B), (0, 0)))
    pcat = jnp.concatenate([sem_proj_w.T, epi_proj_w.T], axis=1)

    gate_call = pl.pallas_call(
        functools.partial(_gate_kernel, T=T, B=B, H=H, L=L, a=a),
        out_shape=[
            jax.ShapeDtypeStruct((TP * B, N), bf16),
            jax.ShapeDtypeStruct((TP * B, 1), f32),
        ],
        scratch_shapes=[
            pltpu.VMEM((TP * B, 2 * H), f32),
            pltpu.VMEM((TP * B, H), f32),
            pltpu.VMEM((TP * B, H), f32),
            pltpu.VMEM((B, H), f32),
            pltpu.VMEM((B, H), f32),
        ],
        compiler_params=pltpu.CompilerParams(
            vmem_limit_bytes=56 * 1024 * 1024),
        name="esn_gate",
    )

    res_call = pl.pallas_call(
        functools.partial(_res_kernel, n_iters=TP),
        grid=(TP,),
        in_specs=[
            pl.BlockSpec((N, N), lambda i: (0, 0)),
            pl.BlockSpec((1, B, N), lambda i: (i, 0, 0)),
            pl.BlockSpec((1, B, 1), lambda i: (i, 0, 0)),
        ],
        out_specs=pl.BlockSpec((B, N), lambda i: (0, 0)),
        out_shape=jax.ShapeDtypeStruct((B, N), f32),
        scratch_shapes=[pltpu.VMEM((B, N), f32)],
        compiler_params=pltpu.CompilerParams(
            dimension_semantics=("arbitrary",),
            vmem_limit_bytes=40 * 1024 * 1024),
        name="esn_reservoir",
    )

    out_call = pl.pallas_call(
        _out_kernel,
        out_shape=jax.ShapeDtypeStruct((B, D), f32),
        name="esn_out",
    )

    def _tail(h_s, h_e):
        return out_call(
            h_s, h_e, sem_readout_w.T, epi_readout_w.T, x[:, -1, :],
            null_gate_w.T, null_gate_b.reshape(1, 1), out_proj_w.T,
            out_proj_b.reshape(1, D), ln_gamma.reshape(1, D),
            ln_beta.reshape(1, D))

    devs = jax.devices()
    if len(devs) >= 2:
        wres2 = jnp.stack([W_res_sem.T.astype(bf16),
                           W_res_epi.T.astype(bf16)])
        winT2 = jnp.stack([W_in_sem.T.astype(bf16),
                           W_in_epi.T.astype(bf16)])
        mesh = Mesh(np.array(devs[:2]), ("r",))

        def _pipeline(wres_sh, winT_sh):
            r_idx = lax.axis_index("r")
            sel = (r_idx == 0).astype(f32).reshape(1, 1)
            u, fire = gate_call(xw, pcat, winT_sh[0], sel)
            h_own = res_call(wres_sh[0], u.reshape(TP, B, N),
                             fire.reshape(TP, B, 1))
            h_pair = lax.all_gather(h_own, "r")
            return _tail(h_pair[0], h_pair[1])

        return jax.shard_map(
            _pipeline, mesh=mesh, in_specs=(P("r"), P("r")),
            out_specs=P(), check_vma=False)(wres2, winT2)

    one = jnp.ones((1, 1), f32)
    zero = jnp.zeros((1, 1), f32)
    u_s, fire = gate_call(xw, pcat, W_in_sem.T.astype(bf16), one)
    u_e, _ = gate_call(xw, pcat, W_in_epi.T.astype(bf16), zero)
    fire3 = fire.reshape(TP, B, 1)
    h_s = res_call(W_res_sem.T.astype(bf16), u_s.reshape(TP, B, N), fire3)
    h_e = res_call(W_res_epi.T.astype(bf16), u_e.reshape(TP, B, N), fire3)
    return _tail(h_s, h_e)
